# baseline (speedup 1.0000x reference)
import functools

import jax
import jax.numpy as jnp
from jax import lax
from jax.experimental import pallas as pl
from jax.experimental.pallas import tpu as pltpu
from jax.experimental.pallas import tpu_sc as plsc

_B = 16384
_D = 64
_R = 1000000
_NW = 32
_L = 16
_CW = 256
_NCH = 122
_NQ = 30
_SPAN = _NCH * _CW
_EXTRA_BASE = _NW * _SPAN
_TAIL_IN = _R - 128
_POSB = 14
_RING = 8

_mesh = plsc.VectorSubcoreMesh(core_axis_name="c", subcore_axis_name="s")


@functools.partial(
    pl.kernel,
    mesh=_mesh,
    out_type=jax.ShapeDtypeStruct((_B * _D,), jnp.float32),
    scratch_types=[
        pltpu.VMEM((_B,), jnp.int32),
        pltpu.VMEM((_B,), jnp.int32),
        pltpu.VMEM((_D, _CW), jnp.float32),
        pltpu.VMEM((_D, _CW), jnp.float32),
        pltpu.VMEM((_D, _CW), jnp.float32),
        pltpu.VMEM((_D, _CW), jnp.float32),
        pltpu.VMEM((_D, 128), jnp.float32),
        pltpu.VMEM((_L,), jnp.int32),
        pltpu.VMEM((_RING * _D,), jnp.float32),
        pltpu.SemaphoreType.DMA,
        pltpu.SemaphoreType.DMA,
        pltpu.SemaphoreType.DMA,
        pltpu.SemaphoreType.DMA,
        pltpu.SemaphoreType.DMA,
    ],
    compiler_params=pltpu.CompilerParams(
        use_tc_tiling_on_sc=True, needs_layout_passes=False),
)
def _sc_stream(idx_hbm, tt_hbm, tail_hbm, out_hbm,
               idx_v, match_v, c0, c1, c2, c3, tail_v, stage_v, ring_v,
               sem0, sem1, sem2, sem3, semc):
    wid = lax.axis_index("s") * 2 + lax.axis_index("c")
    lanes = jnp.arange(_L, dtype=jnp.int32)
    lo = wid * _SPAN
    cbufs = (c0, c1, c2, c3)
    csems = (sem0, sem1, sem2, sem3)

    def popcount(m):
        p = plsc.all_reduce_population_count(m)
        if p.ndim:
            p = lax.reduce_max(p, axes=(0,))
        return p

    pltpu.sync_copy(idx_hbm, idx_v)

    hi = jnp.where(wid == _NW - 1, _R, lo + _SPAN)

    def scan_body(v, cnt):
        rvec = idx_v[pl.ds(v * _L, _L)]
        m = (rvec >= lo) & (rvec < hi)
        pv = ((rvec - lo) << _POSB) | (v * _L + lanes)
        plsc.store_compressed(match_v.at[pl.ds(cnt, _L)], pv, mask=m)
        return cnt + popcount(m)

    n_match = lax.fori_loop(0, _B // _L, scan_body, jnp.int32(0))
    nvec = (n_match + _L - 1) // _L

    def fire(slot, base):
        return pltpu.async_copy(
            tt_hbm.at[:, pl.ds(pl.multiple_of(base, _CW), _CW)],
            cbufs[slot], csems[slot])

    def wait_chunk(slot):
        pltpu.make_async_copy(
            tt_hbm.at[:, pl.ds(0, _CW)], cbufs[slot], csems[slot]).wait()

    def process(cb, filt_lo, filt_hi, col_base, ka):
        plo = filt_lo << _POSB
        phi = filt_hi << _POSB

        def act_body(e, ka):
            svec = stage_v[...]
            p = lax.reduce_sum(jnp.where(lanes == e, svec, 0), axes=(0,))
            col = (p >> _POSB) - col_base
            pos = p & ((1 << _POSB) - 1)
            slot = ka & (_RING - 1)

            @pl.when(ka >= _RING)
            def _():
                pltpu.make_async_copy(
                    ring_v.at[pl.ds(0, _D)], out_hbm.at[pl.ds(0, _D)],
                    semc).wait()

            colvec = jnp.full((_L,), col, jnp.int32)
            base_w = slot * _D
            for g in range(_D // _L):
                vals = plsc.load_gather(cb.at[:, :], [g * _L + lanes, colvec])
                plsc.store_scatter(ring_v.at[pl.ds(0, _RING * _D)], [base_w + g * _L + lanes], vals)
            pltpu.async_copy(
                ring_v.at[pl.ds(base_w, _D)],
                out_hbm.at[pl.ds(pos * _D, _D)], semc)
            return ka + 1

        def mscan_body(v, ka):
            pvec = match_v[pl.ds(v * _L, _L)]
            valid = (v * _L + lanes) < n_match
            m = (pvec >= plo) & (pvec < phi) & valid
            plsc.store_compressed(stage_v.at[pl.ds(0, _L)], pvec, mask=m)
            return lax.fori_loop(0, popcount(m), act_body, ka)

        return lax.fori_loop(0, nvec, mscan_body, ka)

    def drain(k):
        def body(i, c):
            pltpu.make_async_copy(
                ring_v.at[pl.ds(0, _D)], out_hbm.at[pl.ds(0, _D)],
                semc).wait()
            return c

        lax.fori_loop(0, jnp.minimum(k, _RING), body, jnp.int32(0))

    for s in range(3):
        fire(s, lo + s * _CW)

    def quad_body(q, ka):
        for k in range(4):
            cid = q * 4 + k

            @pl.when(cid + 3 < _NCH)
            def _():
                fire((k + 3) % 4, lo + (cid + 3) * _CW)

            wait_chunk(k)
            ka = process(cbufs[k], cid * _CW, (cid + 1) * _CW,
                         cid * _CW, ka)
        return ka

    ka = lax.fori_loop(0, _NQ, quad_body, jnp.int32(0))
    for cid in (_NQ * 4, _NQ * 4 + 1):
        wait_chunk(cid % 4)
        ka = process(cbufs[cid % 4], cid * _CW, (cid + 1) * _CW,
                     cid * _CW, ka)

    @pl.when(wid == _NW - 1)
    def _():
        cp0 = pltpu.async_copy(
            tt_hbm.at[:, pl.ds(_EXTRA_BASE, _CW)], c2, sem2)
        cp1 = pltpu.async_copy(
            tt_hbm.at[:, pl.ds(_EXTRA_BASE + _CW, _CW)], c3, sem3)
        tp = pltpu.async_copy(tail_hbm, tail_v, sem0)
        cp0.wait()
        ka1 = process(c2, _NCH * _CW, _NCH * _CW + _CW, _NCH * _CW, ka)
        cp1.wait()
        ka2 = process(c3, _NCH * _CW + _CW, _NCH * _CW + 2 * _CW,
                      _NCH * _CW + _CW, ka1)
        tp.wait()
        ka3 = process(tail_v, _NCH * _CW + 2 * _CW,
                      _R - _NW * _SPAN + _NCH * _CW,
                      _TAIL_IN - _EXTRA_BASE + _NCH * _CW, ka2)
        drain(ka3)

    @pl.when(wid != _NW - 1)
    def _():
        drain(ka)


def kernel(indexes, table):
    idx = indexes.reshape(_B)
    tt = table.T
    tail = lax.slice(table, (_TAIL_IN, 0), (_R, _D)).T
    flat = _sc_stream(idx, tt, tail)
    return flat.reshape(_B, _D)


# speedup vs baseline: 1.3112x; 1.3112x over previous
import functools

import jax
import jax.numpy as jnp
from jax import lax
from jax.experimental import pallas as pl
from jax.experimental.pallas import tpu as pltpu
from jax.experimental.pallas import tpu_sc as plsc

_B = 16384
_D = 64
_R = 1000000
_NW = 32
_L = 16
_CW = 512
_NCH = 61
_SPAN = _NCH * _CW
_EXTRA_BASE = _NW * _SPAN
_TAIL_IN = _R - 128
_RING = 8
_NVEC = _B // _L

_mesh = plsc.VectorSubcoreMesh(core_axis_name="c", subcore_axis_name="s")


@functools.partial(
    pl.kernel,
    mesh=_mesh,
    out_type=jax.ShapeDtypeStruct((_B * _D,), jnp.float32),
    scratch_types=[
        pltpu.VMEM((_B,), jnp.int32),
        pltpu.VMEM((_B,), jnp.int32),
        pltpu.VMEM((48,), jnp.int32),
        pltpu.VMEM((_D, _CW), jnp.float32),
        pltpu.VMEM((_D, _CW), jnp.float32),
        pltpu.VMEM((_D, 128), jnp.float32),
        pltpu.VMEM((_L,), jnp.int32),
        pltpu.VMEM((_L,), jnp.int32),
        pltpu.VMEM((_RING * _D,), jnp.float32),
        pltpu.SemaphoreType.DMA,
        pltpu.SemaphoreType.DMA,
        pltpu.SemaphoreType.DMA,
    ],
    compiler_params=pltpu.CompilerParams(
        use_tc_tiling_on_sc=True, needs_layout_passes=False),
)
def _sc_stream(srow_hbm, spos_hbm, bounds_hbm, tt_hbm, tail_hbm, out_hbm,
               srow_v, spos_v, bounds_v, c0, c1, tail_v,
               stage_r, stage_p, ring_v, sem0, sem1, semc):
    wid = lax.axis_index("s") * 2 + lax.axis_index("c")
    lanes = jnp.arange(_L, dtype=jnp.int32)
    lo = wid * _SPAN
    cbufs = (c0, c1)
    csems = (sem0, sem1)

    def extract(vec, k):
        return lax.reduce_sum(jnp.where(lanes == k, vec, 0), axes=(0,))

    def popcount(m):
        p = plsc.all_reduce_population_count(m)
        if p.ndim:
            p = lax.reduce_max(p, axes=(0,))
        return p

    pltpu.sync_copy(srow_hbm, srow_v)
    pltpu.sync_copy(spos_hbm, spos_v)
    pltpu.sync_copy(bounds_hbm, bounds_v)
    b0 = bounds_v[pl.ds((wid >> 4) << 4, _L)]
    ps = extract(b0, wid & (_L - 1))
    w1 = wid + 1
    b1 = bounds_v[pl.ds((w1 >> 4) << 4, _L)]
    pe = extract(b1, w1 & (_L - 1))
    pe_vec = (pe + _L - 1) >> 4

    def fire(c, base):
        return pltpu.async_copy(
            tt_hbm.at[:, pl.ds(pl.multiple_of(base, _CW), _CW)],
            cbufs[c % 2], csems[c % 2])

    def wait_chunk(c):
        pltpu.make_async_copy(
            tt_hbm.at[:, pl.ds(0, _CW)], cbufs[c % 2], csems[c % 2]).wait()

    def process(cb, clo, chi, col_base, p, ka):
        def act_body(e, ka):
            r = extract(stage_r[...], e)
            pos = extract(stage_p[...], e)
            col = r - col_base
            slot = ka & (_RING - 1)

            @pl.when(ka >= _RING)
            def _():
                pltpu.make_async_copy(
                    ring_v.at[pl.ds(0, _D)], out_hbm.at[pl.ds(0, _D)],
                    semc).wait()

            colvec = jnp.full((_L,), col, jnp.int32)
            base_w = slot * _D
            for g in range(_D // _L):
                vals = plsc.load_gather(cb.at[:, :],
                                        [g * _L + lanes, colvec])
                plsc.store_scatter(ring_v.at[pl.ds(0, _RING * _D)],
                                   [base_w + g * _L + lanes], vals)
            pltpu.async_copy(
                ring_v.at[pl.ds(base_w, _D)],
                out_hbm.at[pl.ds(pos * _D, _D)], semc)
            return ka + 1

        def cond(state):
            _, _, cont = state
            return cont

        def body(state):
            p, ka, _ = state
            gl = p * _L + lanes
            rvec = srow_v[pl.ds(p * _L, _L)]
            pvec = spos_v[pl.ds(p * _L, _L)]
            seg = (gl >= ps) & (gl < pe)
            m = seg & (rvec >= clo) & (rvec < chi)
            plsc.store_compressed(stage_r.at[pl.ds(0, _L)], rvec, mask=m)
            plsc.store_compressed(stage_p.at[pl.ds(0, _L)], pvec, mask=m)
            ka = lax.fori_loop(0, popcount(m), act_body, ka)
            rmax = lax.reduce_max(jnp.where(seg, rvec, -1), axes=(0,))
            adv = rmax < chi
            pn = p + adv.astype(jnp.int32)
            return pn, ka, adv & (pn < pe_vec)

        p, ka, _ = lax.while_loop(cond, body, (p, ka, p < pe_vec))
        return p, ka

    def drain(k):
        def body(i, c):
            pltpu.make_async_copy(
                ring_v.at[pl.ds(0, _D)], out_hbm.at[pl.ds(0, _D)],
                semc).wait()
            return c

        lax.fori_loop(0, jnp.minimum(k, _RING), body, jnp.int32(0))

    fire(0, lo)

    def pair_body(q, state):
        p, ka = state
        a = 2 * q
        fire(1, lo + (a + 1) * _CW)
        wait_chunk(0)
        p, ka = process(c0, lo + a * _CW, lo + (a + 1) * _CW,
                        lo + a * _CW, p, ka)
        fire(0, lo + (a + 2) * _CW)
        wait_chunk(1)
        p, ka = process(c1, lo + (a + 1) * _CW, lo + (a + 2) * _CW,
                        lo + (a + 1) * _CW, p, ka)
        return p, ka

    p, ka = lax.fori_loop(0, _NCH // 2, pair_body,
                          (ps >> 4, jnp.int32(0)))
    wait_chunk(0)
    p, ka = process(c0, lo + (_NCH - 1) * _CW, lo + _NCH * _CW,
                    lo + (_NCH - 1) * _CW, p, ka)

    @pl.when(wid == _NW - 1)
    def _():
        cp = pltpu.async_copy(
            tt_hbm.at[:, pl.ds(_EXTRA_BASE, _CW)], c0, sem0)
        tp = pltpu.async_copy(tail_hbm, tail_v, sem1)
        cp.wait()
        p1, ka1 = process(c0, _EXTRA_BASE, _EXTRA_BASE + _CW,
                          _EXTRA_BASE, p, ka)
        tp.wait()
        _, ka2 = process(tail_v, _EXTRA_BASE + _CW, _R, _TAIL_IN, p1, ka1)
        drain(ka2)

    @pl.when(wid != _NW - 1)
    def _():
        drain(ka)


def kernel(indexes, table):
    idx = indexes.reshape(_B)
    srow, spos = lax.sort_key_val(idx, jnp.arange(_B, dtype=jnp.int32))
    edges = jnp.concatenate([
        jnp.arange(_NW, dtype=jnp.int32) * _SPAN,
        jnp.array([_R], dtype=jnp.int32)])
    bounds = jnp.searchsorted(srow, edges).astype(jnp.int32)
    bounds = jnp.pad(bounds, (0, 48 - _NW - 1))
    tt = table.T
    tail = lax.slice(table, (_TAIL_IN, 0), (_R, _D)).T
    flat = _sc_stream(srow, spos, bounds, tt, tail)
    return flat.reshape(_B, _D)


# speedup vs baseline: 1.3414x; 1.0230x over previous
import functools

import jax
import jax.numpy as jnp
from jax import lax
from jax.experimental import pallas as pl
from jax.experimental.pallas import tpu as pltpu
from jax.experimental.pallas import tpu_sc as plsc

_B = 16384
_D = 64
_R = 1000000
_NW = 32
_L = 16
_CW = 512
_NCH = 61
_SPAN = _NCH * _CW
_EXTRA_BASE = _NW * _SPAN
_TAIL_IN = _R - 128
_POSB = 14
_RING = 8
_GCH = 8

_mesh = plsc.VectorSubcoreMesh(core_axis_name="c", subcore_axis_name="s")


@functools.partial(
    pl.kernel,
    mesh=_mesh,
    out_type=jax.ShapeDtypeStruct((_B * _D,), jnp.float32),
    scratch_types=[
        pltpu.VMEM((_B,), jnp.int32),
        pltpu.VMEM((_B,), jnp.int32),
        pltpu.VMEM((_D, _CW), jnp.float32),
        pltpu.VMEM((_D, _CW), jnp.float32),
        pltpu.VMEM((_D, 128), jnp.float32),
        pltpu.VMEM((_L,), jnp.int32),
        pltpu.VMEM((_RING * _D,), jnp.float32),
        pltpu.SemaphoreType.DMA,
        pltpu.SemaphoreType.DMA,
        pltpu.SemaphoreType.DMA,
    ],
    compiler_params=pltpu.CompilerParams(
        use_tc_tiling_on_sc=True, needs_layout_passes=False),
)
def _sc_stream(idx_hbm, tt_hbm, tail_hbm, out_hbm,
               buf_a, match_v, c0, c1, tail_v, stage_v, ring_v,
               sem0, sem1, semc):
    wid = lax.axis_index("s") * 2 + lax.axis_index("c")
    lanes = jnp.arange(_L, dtype=jnp.int32)
    lo = wid * _SPAN
    cbufs = (c0, c1)
    csems = (sem0, sem1)

    def extract(vec, k):
        return lax.reduce_sum(jnp.where(lanes == k, vec, 0), axes=(0,))

    def popcount(m):
        p = plsc.all_reduce_population_count(m)
        if p.ndim:
            p = lax.reduce_max(p, axes=(0,))
        return p

    pltpu.sync_copy(idx_hbm, buf_a)

    hi = jnp.where(wid == _NW - 1, _R, lo + _SPAN)

    def scan_body(v, cnt):
        rvec = buf_a[pl.ds(v * _L, _L)]
        m = (rvec >= lo) & (rvec < hi)
        pv = ((rvec - lo) << _POSB) | (v * _L + lanes)
        plsc.store_compressed(match_v.at[pl.ds(cnt, _L)], pv, mask=m)
        return cnt + popcount(m)

    n_match = lax.fori_loop(0, _B // _L, scan_body, jnp.int32(0))
    nvec = (n_match + _L - 1) >> 4

    def fire(slot, base):
        return pltpu.async_copy(
            tt_hbm.at[:, pl.ds(pl.multiple_of(base, _CW), _CW)],
            cbufs[slot], csems[slot])

    def wait_chunk(slot):
        pltpu.make_async_copy(
            tt_hbm.at[:, pl.ds(0, _CW)], cbufs[slot], csems[slot]).wait()

    def make_process(src_v, n_src):
        def process(cb, filt_lo, filt_hi, col_base, ka):
            plo = filt_lo << _POSB
            phi = filt_hi << _POSB

            def act_body(e, ka):
                p = extract(stage_v[...], e)
                col = (p >> _POSB) - col_base
                pos = p & ((1 << _POSB) - 1)
                slot = ka & (_RING - 1)

                @pl.when(ka >= _RING)
                def _():
                    pltpu.make_async_copy(
                        ring_v.at[pl.ds(0, _D)], out_hbm.at[pl.ds(0, _D)],
                        semc).wait()

                colvec = jnp.full((_L,), col, jnp.int32)
                base_w = slot * _D
                for g in range(_D // _L):
                    vals = plsc.load_gather(cb.at[:, :],
                                            [g * _L + lanes, colvec])
                    plsc.store_scatter(ring_v.at[pl.ds(0, _RING * _D)],
                                       [base_w + g * _L + lanes], vals)
                pltpu.async_copy(
                    ring_v.at[pl.ds(base_w, _D)],
                    out_hbm.at[pl.ds(pos * _D, _D)], semc)
                return ka + 1

            def mscan_body(v, ka):
                pvec = src_v[pl.ds(v * _L, _L)]
                valid = (v * _L + lanes) < n_src
                m = (pvec >= plo) & (pvec < phi) & valid
                plsc.store_compressed(stage_v.at[pl.ds(0, _L)], pvec,
                                      mask=m)
                return lax.fori_loop(0, popcount(m), act_body, ka)

            return lax.fori_loop(0, (n_src + _L - 1) >> 4, mscan_body, ka)

        return process

    process_full = make_process(match_v, n_match)

    def drain(k):
        def body(i, c):
            pltpu.make_async_copy(
                ring_v.at[pl.ds(0, _D)], out_hbm.at[pl.ds(0, _D)],
                semc).wait()
            return c

        lax.fori_loop(0, jnp.minimum(k, _RING), body, jnp.int32(0))

    fire(0, lo)
    ka = jnp.int32(0)
    for grp in range((_NCH + _GCH - 1) // _GCH):
        gc0 = grp * _GCH
        gc1 = min(gc0 + _GCH, _NCH)
        glo = (gc0 * _CW) << _POSB
        ghi = (gc1 * _CW) << _POSB

        def gfilt_body(v, cnt, glo=glo, ghi=ghi):
            pvec = match_v[pl.ds(v * _L, _L)]
            valid = (v * _L + lanes) < n_match
            m = (pvec >= glo) & (pvec < ghi) & valid
            plsc.store_compressed(buf_a.at[pl.ds(cnt, _L)], pvec, mask=m)
            return cnt + popcount(m)

        ng = lax.fori_loop(0, nvec, gfilt_body, jnp.int32(0))
        process_g = make_process(buf_a, ng)

        def pair_body(q, ka, gc0=gc0, process_g=process_g):
            a = gc0 + 2 * q
            fire(1, lo + (a + 1) * _CW)
            wait_chunk(0)
            ka = process_g(c0, a * _CW, (a + 1) * _CW, a * _CW, ka)
            fire(0, lo + (a + 2) * _CW)
            wait_chunk(1)
            ka = process_g(c1, (a + 1) * _CW, (a + 2) * _CW,
                           (a + 1) * _CW, ka)
            return ka

        ka = lax.fori_loop(0, (gc1 - gc0) // 2, pair_body, ka)

    wait_chunk(0)
    ka = process_full(c0, (_NCH - 1) * _CW, _NCH * _CW,
                      (_NCH - 1) * _CW, ka)

    @pl.when(wid == _NW - 1)
    def _():
        cp = pltpu.async_copy(
            tt_hbm.at[:, pl.ds(_EXTRA_BASE, _CW)], c1, sem1)
        tp = pltpu.async_copy(tail_hbm, tail_v, sem0)
        cp.wait()
        ka1 = process_full(c1, _NCH * _CW, _NCH * _CW + _CW,
                           _NCH * _CW, ka)
        tp.wait()
        ka2 = process_full(tail_v, _NCH * _CW + _CW,
                           _R - _NW * _SPAN + _NCH * _CW,
                           _TAIL_IN - _EXTRA_BASE + _NCH * _CW, ka1)
        drain(ka2)

    @pl.when(wid != _NW - 1)
    def _():
        drain(ka)


def kernel(indexes, table):
    idx = indexes.reshape(_B)
    tt = table.T
    tail = lax.slice(table, (_TAIL_IN, 0), (_R, _D)).T
    flat = _sc_stream(idx, tt, tail)
    return flat.reshape(_B, _D)


# speedup vs baseline: 1.3784x; 1.0276x over previous
import functools

import jax
import jax.numpy as jnp
from jax import lax
from jax.experimental import pallas as pl
from jax.experimental.pallas import tpu as pltpu
from jax.experimental.pallas import tpu_sc as plsc

_B = 16384
_D = 64
_R = 1000000
_NW = 32
_L = 16
_CW = 512
_NCH = 61
_SPAN = _NCH * _CW
_EXTRA_BASE = _NW * _SPAN
_TAIL_IN = _R - 128
_POSB = 14
_RING = 8
_GCH = 8

_mesh = plsc.VectorSubcoreMesh(core_axis_name="c", subcore_axis_name="s")


@functools.partial(
    pl.kernel,
    mesh=_mesh,
    out_type=jax.ShapeDtypeStruct((_B * _D,), jnp.float32),
    scratch_types=[
        pltpu.VMEM((_B,), jnp.int32),
        pltpu.VMEM((_B,), jnp.int32),
        pltpu.VMEM((_D, _CW), jnp.float32),
        pltpu.VMEM((_D, _CW), jnp.float32),
        pltpu.VMEM((_D, 128), jnp.float32),
        pltpu.VMEM((_L,), jnp.int32),
        pltpu.VMEM((256,), jnp.int32),
        pltpu.VMEM((_RING * _D,), jnp.float32),
        pltpu.SemaphoreType.DMA,
        pltpu.SemaphoreType.DMA,
        pltpu.SemaphoreType.DMA,
    ],
    compiler_params=pltpu.CompilerParams(
        use_tc_tiling_on_sc=True, needs_layout_passes=False),
)
def _sc_stream(idx_hbm, tt_hbm, tail_hbm, out_hbm,
               buf_a, match_v, c0, c1, tail_v, stage_v, occ_v, ring_v,
               sem0, sem1, semc):
    wid = lax.axis_index("s") * 2 + lax.axis_index("c")
    lanes = jnp.arange(_L, dtype=jnp.int32)
    lo = wid * _SPAN
    cbufs = (c0, c1)
    csems = (sem0, sem1)

    def extract(vec, k):
        return lax.reduce_sum(jnp.where(lanes == k, vec, 0), axes=(0,))

    def popcount(m):
        p = plsc.all_reduce_population_count(m)
        if p.ndim:
            p = lax.reduce_max(p, axes=(0,))
        return p

    pltpu.sync_copy(idx_hbm, buf_a)

    zeros16 = jnp.zeros((_L,), jnp.int32)
    for z in range(256 // _L):
        occ_v[pl.ds(z * _L, _L)] = zeros16

    hi = jnp.where(wid == _NW - 1, _R, lo + _SPAN)
    ones16 = jnp.ones((_L,), jnp.int32)

    def scan_body(v, cnt):
        rvec = buf_a[pl.ds(v * _L, _L)]
        m = (rvec >= lo) & (rvec < hi)
        pv = ((rvec - lo) << _POSB) | (v * _L + lanes)
        plsc.store_compressed(match_v.at[pl.ds(cnt, _L)], pv, mask=m)
        plsc.store_scatter(occ_v.at[pl.ds(0, 256)],
                           [((rvec - lo) >> 7) & 255], ones16, mask=m)
        return cnt + popcount(m)

    n_match = lax.fori_loop(0, _B // _L, scan_body, jnp.int32(0))
    nvec = (n_match + _L - 1) >> 4

    def occupied(tc):
        ovec = occ_v[pl.ds((tc >> 4) << 4, _L)]
        return extract(ovec, tc & (_L - 1)) > 0

    def fire(slot, c):
        base = lo + c * _CW
        for k in range(_CW // 128):
            tc = c * (_CW // 128) + k

            @pl.when(occupied(tc))
            def _(k=k):
                pltpu.async_copy(
                    tt_hbm.at[:, pl.ds(
                        pl.multiple_of(base + k * 128, 128), 128)],
                    cbufs[slot].at[:, pl.ds(k * 128, 128)], csems[slot])

    def wait_chunk(slot, c):
        for k in range(_CW // 128):
            tc = c * (_CW // 128) + k

            @pl.when(occupied(tc))
            def _(k=k):
                pltpu.make_async_copy(
                    tt_hbm.at[:, pl.ds(0, 128)],
                    cbufs[slot].at[:, pl.ds(k * 128, 128)],
                    csems[slot]).wait()

    def make_process(src_v, n_src):
        def process(cb, filt_lo, filt_hi, col_base, ka):
            plo = filt_lo << _POSB
            phi = filt_hi << _POSB

            def act_body(e, ka):
                p = extract(stage_v[...], e)
                col = (p >> _POSB) - col_base
                pos = p & ((1 << _POSB) - 1)
                slot = ka & (_RING - 1)

                @pl.when(ka >= _RING)
                def _():
                    pltpu.make_async_copy(
                        ring_v.at[pl.ds(0, _D)], out_hbm.at[pl.ds(0, _D)],
                        semc).wait()

                colvec = jnp.full((_L,), col, jnp.int32)
                base_w = slot * _D
                for g in range(_D // _L):
                    vals = plsc.load_gather(cb.at[:, :],
                                            [g * _L + lanes, colvec])
                    plsc.store_scatter(ring_v.at[pl.ds(0, _RING * _D)],
                                       [base_w + g * _L + lanes], vals)
                pltpu.async_copy(
                    ring_v.at[pl.ds(base_w, _D)],
                    out_hbm.at[pl.ds(pos * _D, _D)], semc)
                return ka + 1

            def mscan_body(v, ka):
                pvec = src_v[pl.ds(v * _L, _L)]
                valid = (v * _L + lanes) < n_src
                m = (pvec >= plo) & (pvec < phi) & valid
                plsc.store_compressed(stage_v.at[pl.ds(0, _L)], pvec,
                                      mask=m)
                return lax.fori_loop(0, popcount(m), act_body, ka)

            return lax.fori_loop(0, (n_src + _L - 1) >> 4, mscan_body, ka)

        return process

    process_full = make_process(match_v, n_match)

    def drain(k):
        def body(i, c):
            pltpu.make_async_copy(
                ring_v.at[pl.ds(0, _D)], out_hbm.at[pl.ds(0, _D)],
                semc).wait()
            return c

        lax.fori_loop(0, jnp.minimum(k, _RING), body, jnp.int32(0))

    fire(0, 0)
    ka = jnp.int32(0)
    for grp in range((_NCH + _GCH - 1) // _GCH):
        gc0 = grp * _GCH
        gc1 = min(gc0 + _GCH, _NCH)
        glo = (gc0 * _CW) << _POSB
        ghi = (gc1 * _CW) << _POSB

        def gfilt_body(v, cnt, glo=glo, ghi=ghi):
            pvec = match_v[pl.ds(v * _L, _L)]
            valid = (v * _L + lanes) < n_match
            m = (pvec >= glo) & (pvec < ghi) & valid
            plsc.store_compressed(buf_a.at[pl.ds(cnt, _L)], pvec, mask=m)
            return cnt + popcount(m)

        ng = lax.fori_loop(0, nvec, gfilt_body, jnp.int32(0))
        process_g = make_process(buf_a, ng)

        def pair_body(q, ka, gc0=gc0, process_g=process_g):
            a = gc0 + 2 * q
            fire(1, a + 1)
            wait_chunk(0, a)
            ka = process_g(c0, a * _CW, (a + 1) * _CW, a * _CW, ka)
            fire(0, a + 2)
            wait_chunk(1, a + 1)
            ka = process_g(c1, (a + 1) * _CW, (a + 2) * _CW,
                           (a + 1) * _CW, ka)
            return ka

        ka = lax.fori_loop(0, (gc1 - gc0) // 2, pair_body, ka)

    wait_chunk(0, _NCH - 1)
    ka = process_full(c0, (_NCH - 1) * _CW, _NCH * _CW,
                      (_NCH - 1) * _CW, ka)

    @pl.when(wid == _NW - 1)
    def _():
        cp = pltpu.async_copy(
            tt_hbm.at[:, pl.ds(_EXTRA_BASE, _CW)], c1, sem1)
        tp = pltpu.async_copy(tail_hbm, tail_v, sem0)
        cp.wait()
        ka1 = process_full(c1, _NCH * _CW, _NCH * _CW + _CW,
                           _NCH * _CW, ka)
        tp.wait()
        ka2 = process_full(tail_v, _NCH * _CW + _CW,
                           _R - _NW * _SPAN + _NCH * _CW,
                           _TAIL_IN - _EXTRA_BASE + _NCH * _CW, ka1)
        drain(ka2)

    @pl.when(wid != _NW - 1)
    def _():
        drain(ka)


def kernel(indexes, table):
    idx = indexes.reshape(_B)
    tt = table.T
    tail = lax.slice(table, (_TAIL_IN, 0), (_R, _D)).T
    flat = _sc_stream(idx, tt, tail)
    return flat.reshape(_B, _D)
